# baseline (speedup 1.0000x reference)
import functools

import jax
import jax.numpy as jnp
from jax import lax
from jax.experimental import pallas as pl
from jax.experimental.pallas import tpu as pltpu
from jax.experimental.pallas import tpu_sc as plsc

VOCAB = 1000012
B = 16384
NF = 26
D = 16

NC = 2
NS = 16
NW = NC * NS
BPW = B // NW
IPW = BPW * NF

NFULL = VOCAB // 128
NTAIL = VOCAB - NFULL * 128
NSLOT = NFULL // NW + 1

CH = 16
RPC = CH * NF
GROWS = 104
NCHUNK = BPW // CH


def _lane_broadcast(vec, lane):
    idx = jnp.full((16, 1), lane, jnp.int32)
    dnums = lax.GatherDimensionNumbers(
        offset_dims=(), collapsed_slice_dims=(0,), start_index_map=(0,))
    return lax.gather(vec, idx, dnums, (1,),
                      mode=lax.GatherScatterMode.PROMISE_IN_BOUNDS)


def _transpose_block(in_ref, out_ref):
    iota = lax.broadcasted_iota(jnp.int32, (16,), 0)
    for j in range(128):
        col = (iota + j) & 127
        vals = plsc.load_gather(in_ref, [iota, col])
        plsc.store_scatter(out_ref, [col * 16 + iota], vals)


def _stage1_body(tview_hbm, tail_hbm, out_hbm, in_v0, in_v1, in_v2, in_v3,
                 out_v0, out_v1, out_v2, out_v3, sem_in, sem_out):
    wid = lax.axis_index("s") * NC + lax.axis_index("c")
    in_bufs = (in_v0, in_v1, in_v2, in_v3)
    out_bufs = (out_v0, out_v1, out_v2, out_v3)

    def issue_in(t, b):
        cid = t * NW + wid
        pltpu.async_copy(tview_hbm.at[:, pl.ds(cid * 128, 128)],
                         in_bufs[b], sem_in.at[b])

    def wait_in(b):
        pltpu.make_async_copy(tview_hbm.at[:, pl.ds(0, 128)],
                              in_bufs[b], sem_in.at[b]).wait()

    def issue_out(cid, b):
        pltpu.async_copy(out_bufs[b],
                         out_hbm.at[pl.ds(cid * 2048, 2048)], sem_out.at[b])

    def wait_out(b):
        pltpu.make_async_copy(out_bufs[b],
                              out_hbm.at[pl.ds(0, 2048)], sem_out.at[b]).wait()

    for b in range(4):
        issue_in(b, b)

    def loop_body(tt, carry):
        for b in range(4):
            t = 4 * tt + b
            cid = t * NW + wid

            @pl.when(cid < NFULL)
            def _():
                wait_in(b)

                @pl.when(t >= 4)
                def _():
                    wait_out(b)

                _transpose_block(in_bufs[b], out_bufs[b])
                issue_out(cid, b)

            @pl.when((t + 4) * NW + wid < NFULL)
            def _():
                issue_in(t + 4, b)
        return carry

    lax.fori_loop(0, (NSLOT + 3) // 4, loop_body, 0)
    for b in range(4):
        wait_out(b)

    @pl.when(wid == NW - 1)
    def _():
        pltpu.sync_copy(tail_hbm, in_v0)
        _transpose_block(in_v0, out_v0)
        pltpu.sync_copy(out_v0.at[pl.ds(0, NTAIL * 16)],
                        out_hbm.at[pl.ds(NFULL * 2048, NTAIL * 16)])


def _stage2_body(ids_hbm, rat_hbm, table_hbm, bias_hbm, out_hbm,
                 idx_v, rat_v, rows_v, out_v, bias_v, sems):
    wid = lax.axis_index("s") * NC + lax.axis_index("c")

    pltpu.sync_copy(ids_hbm.at[wid], idx_v)
    pltpu.sync_copy(rat_hbm.at[wid], rat_v)
    pltpu.sync_copy(bias_hbm, bias_v)
    bvec = bias_v[...]
    zvec = jnp.zeros((16,), jnp.float32)

    def issue(chunk, buf):
        for g in range(4):
            pltpu.async_copy(
                table_hbm.at[idx_v.at[4 * chunk + g]],
                rows_v.at[pl.ds(buf * RPC + g * GROWS, GROWS)],
                sems.at[buf])

    def drain(buf):
        for g in range(4):
            pltpu.make_async_copy(
                table_hbm.at[idx_v.at[g]],
                rows_v.at[pl.ds(buf * RPC + g * GROWS, GROWS)],
                sems.at[buf]).wait()

    issue(0, 0)
    issue(1, 1)

    def compute(chunk, buf):
        base = buf * RPC

        def sample_body(s, carry2):
            gbase = (chunk * CH + s) * 32
            rv0 = rat_v[pl.ds(gbase, 16)]
            rv1 = rat_v[pl.ds(gbase + 16, 16)]
            accs = [bvec, zvec, zvec, zvec]
            for f in range(NF):
                rv = rv0 if f < 16 else rv1
                rb = _lane_broadcast(rv, f % 16)
                row = rows_v[base + s * NF + f]
                accs[f % 4] = accs[f % 4] + row * rb
            out_v[chunk * CH + s] = (accs[0] + accs[1]) + (accs[2] + accs[3])
            return carry2

        lax.fori_loop(0, CH, sample_body, 0)

    def loop_body(tt, carry):
        for b in range(2):
            c = 2 * tt + b
            drain(b)
            compute(c, b)

            @pl.when(c + 2 < NCHUNK)
            def _():
                issue(c + 2, b)
        return carry

    lax.fori_loop(0, NCHUNK // 2, loop_body, 0)
    pltpu.sync_copy(out_v, out_hbm.at[wid])


def kernel(feature_ids, feature_ratings, fc_weight, bias):
    tview = fc_weight.T
    tail128 = jnp.pad(fc_weight[NFULL * 128:].T, ((0, 0), (0, 128 - NTAIL)))

    stage1 = functools.partial(
        pl.kernel,
        out_type=jax.ShapeDtypeStruct((VOCAB * D,), jnp.float32),
        mesh=plsc.VectorSubcoreMesh(core_axis_name="c", subcore_axis_name="s"),
        compiler_params=pltpu.CompilerParams(
            use_tc_tiling_on_sc=True, needs_layout_passes=False),
        scratch_types=[
            pltpu.VMEM((16, 128), jnp.float32),
            pltpu.VMEM((16, 128), jnp.float32),
            pltpu.VMEM((16, 128), jnp.float32),
            pltpu.VMEM((16, 128), jnp.float32),
            pltpu.VMEM((2048,), jnp.float32),
            pltpu.VMEM((2048,), jnp.float32),
            pltpu.VMEM((2048,), jnp.float32),
            pltpu.VMEM((2048,), jnp.float32),
            pltpu.SemaphoreType.DMA((2,)),
            pltpu.SemaphoreType.DMA((2,)),
        ],
    )(_stage1_body)

    table_rm = stage1(tview, tail128).reshape(VOCAB, D)

    ids3 = feature_ids.reshape(NW, 4 * NCHUNK, GROWS)
    rat2 = jnp.pad(feature_ratings, ((0, 0), (0, 32 - NF))).reshape(NW, BPW * 32)

    stage2 = functools.partial(
        pl.kernel,
        out_type=jax.ShapeDtypeStruct((NW, BPW, D), jnp.float32),
        mesh=plsc.VectorSubcoreMesh(core_axis_name="c", subcore_axis_name="s"),
        compiler_params=pltpu.CompilerParams(use_tc_tiling_on_sc=False),
        scratch_types=[
            pltpu.VMEM((4 * NCHUNK, GROWS), jnp.int32),
            pltpu.VMEM((BPW * 32,), jnp.float32),
            pltpu.VMEM((2 * RPC, D), jnp.float32),
            pltpu.VMEM((BPW, D), jnp.float32),
            pltpu.VMEM((D,), jnp.float32),
            pltpu.SemaphoreType.DMA((2,)),
        ],
    )(_stage2_body)

    out = stage2(ids3, rat2, table_rm, bias)
    return out.reshape(B, D)


# speedup vs baseline: 1.0003x; 1.0003x over previous
import functools

import jax
import jax.numpy as jnp
from jax import lax
from jax.experimental import pallas as pl
from jax.experimental.pallas import tpu as pltpu
from jax.experimental.pallas import tpu_sc as plsc

VOCAB = 1000012
B = 16384
NF = 26
D = 16

NC = 2
NS = 16
NW = NC * NS
BPW = B // NW
IPW = BPW * NF

NFULL = VOCAB // 128
NTAIL = VOCAB - NFULL * 128
NSLOT = NFULL // NW + 1

CH = 16
RPC = CH * NF
GROWS = 104
NCHUNK = BPW // CH


def _lane_broadcast(vec, lane):
    idx = jnp.full((16, 1), lane, jnp.int32)
    dnums = lax.GatherDimensionNumbers(
        offset_dims=(), collapsed_slice_dims=(0,), start_index_map=(0,))
    return lax.gather(vec, idx, dnums, (1,),
                      mode=lax.GatherScatterMode.PROMISE_IN_BOUNDS)


def _transpose_block(in_ref, out_ref):
    iota = lax.broadcasted_iota(jnp.int32, (16,), 0)
    for j in range(128):
        col = (iota + j) & 127
        vals = plsc.load_gather(in_ref, [iota, col])
        plsc.store_scatter(out_ref, [col * 16 + iota], vals)


def _stage1_body(tview_hbm, tail_hbm, out_hbm, in_v0, in_v1, in_v2, in_v3,
                 out_v0, out_v1, out_v2, out_v3, sem_in, sem_out):
    wid = lax.axis_index("s") * NC + lax.axis_index("c")
    in_bufs = (in_v0, in_v1, in_v2, in_v3)
    out_bufs = (out_v0, out_v1, out_v2, out_v3)

    def issue_in(t, b):
        cid = t * NW + wid
        pltpu.async_copy(tview_hbm.at[:, pl.ds(cid * 128, 128)],
                         in_bufs[b], sem_in.at[b])

    def wait_in(b):
        pltpu.make_async_copy(tview_hbm.at[:, pl.ds(0, 128)],
                              in_bufs[b], sem_in.at[b]).wait()

    def issue_out(cid, b):
        pltpu.async_copy(out_bufs[b],
                         out_hbm.at[pl.ds(cid * 2048, 2048)], sem_out.at[b])

    def wait_out(b):
        pltpu.make_async_copy(out_bufs[b],
                              out_hbm.at[pl.ds(0, 2048)], sem_out.at[b]).wait()

    for b in range(4):
        issue_in(b, b)

    def loop_body(tt, carry):
        for b in range(4):
            t = 4 * tt + b
            cid = t * NW + wid

            @pl.when(cid < NFULL)
            def _():
                wait_in(b)

                @pl.when(t >= 4)
                def _():
                    wait_out(b)

                _transpose_block(in_bufs[b], out_bufs[b])
                issue_out(cid, b)

            @pl.when((t + 4) * NW + wid < NFULL)
            def _():
                issue_in(t + 4, b)
        return carry

    lax.fori_loop(0, (NSLOT + 3) // 4, loop_body, 0)
    for b in range(4):
        wait_out(b)

    @pl.when(wid == NW - 1)
    def _():
        pltpu.sync_copy(tail_hbm, in_v0)
        _transpose_block(in_v0, out_v0)
        pltpu.sync_copy(out_v0.at[pl.ds(0, NTAIL * 16)],
                        out_hbm.at[pl.ds(NFULL * 2048, NTAIL * 16)])


def _stage2_body(ids_hbm, rat_hbm, table_hbm, bias_hbm, out_hbm,
                 idx_v, rat_v, rows_v, out_v, bias_v, sems):
    wid = lax.axis_index("s") * NC + lax.axis_index("c")

    pltpu.sync_copy(ids_hbm.at[wid], idx_v)
    pltpu.sync_copy(rat_hbm.at[wid], rat_v)
    pltpu.sync_copy(bias_hbm, bias_v)
    bvec = bias_v[...]
    zvec = jnp.zeros((16,), jnp.float32)

    def issue(chunk, buf):
        for g in range(4):
            pltpu.async_copy(
                table_hbm.at[idx_v.at[4 * chunk + g]],
                rows_v.at[pl.ds(buf * RPC + g * GROWS, GROWS)],
                sems.at[buf])

    def drain(buf):
        for g in range(4):
            pltpu.make_async_copy(
                table_hbm.at[idx_v.at[g]],
                rows_v.at[pl.ds(buf * RPC + g * GROWS, GROWS)],
                sems.at[buf]).wait()

    issue(0, 0)
    issue(1, 1)

    def compute(chunk, buf):
        base = buf * RPC

        def sample_body(s, carry2):
            gbase = (chunk * CH + s) * 32
            rv0 = rat_v[pl.ds(gbase, 16)]
            rv1 = rat_v[pl.ds(gbase + 16, 16)]
            accs = [bvec, zvec, zvec, zvec]
            for f in range(NF):
                rv = rv0 if f < 16 else rv1
                rb = _lane_broadcast(rv, f % 16)
                row = rows_v[base + s * NF + f]
                accs[f % 4] = accs[f % 4] + row * rb
            out_v[chunk * CH + s] = (accs[0] + accs[1]) + (accs[2] + accs[3])
            return carry2

        lax.fori_loop(0, CH, sample_body, 0)

    def loop_body(tt, carry):
        for b in range(2):
            c = 2 * tt + b
            drain(b)
            compute(c, b)

            @pl.when(c + 2 < NCHUNK)
            def _():
                issue(c + 2, b)
        return carry

    lax.fori_loop(0, NCHUNK // 2, loop_body, 0)
    pltpu.sync_copy(out_v, out_hbm.at[wid])


def kernel(feature_ids, feature_ratings, fc_weight, bias):
    tview = fc_weight.T
    tail128 = jnp.pad(fc_weight[NFULL * 128:].T, ((0, 0), (0, 128 - NTAIL)))

    stage1 = functools.partial(
        pl.kernel,
        out_type=jax.ShapeDtypeStruct((VOCAB * D,), jnp.float32),
        mesh=plsc.VectorSubcoreMesh(core_axis_name="c", subcore_axis_name="s"),
        compiler_params=pltpu.CompilerParams(
            use_tc_tiling_on_sc=True, needs_layout_passes=False),
        scratch_types=[
            pltpu.VMEM((16, 128), jnp.float32),
            pltpu.VMEM((16, 128), jnp.float32),
            pltpu.VMEM((16, 128), jnp.float32),
            pltpu.VMEM((16, 128), jnp.float32),
            pltpu.VMEM((2048,), jnp.float32),
            pltpu.VMEM((2048,), jnp.float32),
            pltpu.VMEM((2048,), jnp.float32),
            pltpu.VMEM((2048,), jnp.float32),
            pltpu.SemaphoreType.DMA((4,)),
            pltpu.SemaphoreType.DMA((4,)),
        ],
    )(_stage1_body)

    table_rm = stage1(tview, tail128).reshape(VOCAB, D)

    ids3 = feature_ids.reshape(NW, 4 * NCHUNK, GROWS)
    rat2 = jnp.pad(feature_ratings, ((0, 0), (0, 32 - NF))).reshape(NW, BPW * 32)

    stage2 = functools.partial(
        pl.kernel,
        out_type=jax.ShapeDtypeStruct((NW, BPW, D), jnp.float32),
        mesh=plsc.VectorSubcoreMesh(core_axis_name="c", subcore_axis_name="s"),
        compiler_params=pltpu.CompilerParams(use_tc_tiling_on_sc=False),
        scratch_types=[
            pltpu.VMEM((4 * NCHUNK, GROWS), jnp.int32),
            pltpu.VMEM((BPW * 32,), jnp.float32),
            pltpu.VMEM((2 * RPC, D), jnp.float32),
            pltpu.VMEM((BPW, D), jnp.float32),
            pltpu.VMEM((D,), jnp.float32),
            pltpu.SemaphoreType.DMA((2,)),
        ],
    )(_stage2_body)

    out = stage2(ids3, rat2, table_rm, bias)
    return out.reshape(B, D)


# speedup vs baseline: 1.6704x; 1.6699x over previous
import functools

import jax
import jax.numpy as jnp
from jax import lax
from jax.experimental import pallas as pl
from jax.experimental.pallas import tpu as pltpu
from jax.experimental.pallas import tpu_sc as plsc

VOCAB = 1000012
B = 16384
NF = 26
D = 16

NC = 2
NS = 16
NW = NC * NS
BPW = B // NW
IPW = BPW * NF

NFULL = VOCAB // 128
NTAIL = VOCAB - NFULL * 128
NSLOT = NFULL // NW + 1

CH = 16
RPC = CH * NF
GROWS = 104
NCHUNK = BPW // CH


def _lane_broadcast(vec, lane):
    idx = jnp.full((16, 1), lane, jnp.int32)
    dnums = lax.GatherDimensionNumbers(
        offset_dims=(), collapsed_slice_dims=(0,), start_index_map=(0,))
    return lax.gather(vec, idx, dnums, (1,),
                      mode=lax.GatherScatterMode.PROMISE_IN_BOUNDS)


def _transpose_block(in_ref, out_ref):
    iota = lax.broadcasted_iota(jnp.int32, (16,), 0)

    @plsc.parallel_loop(0, 128, unroll=16)
    def _(j):
        col = (iota + j) & 127
        vals = plsc.load_gather(in_ref, [iota, col])
        plsc.store_scatter(out_ref, [col * 16 + iota], vals)


def _stage1_body(tview_hbm, tail_hbm, out_hbm, in_v0, in_v1, in_v2, in_v3,
                 out_v0, out_v1, out_v2, out_v3, sem_in, sem_out):
    wid = lax.axis_index("s") * NC + lax.axis_index("c")
    in_bufs = (in_v0, in_v1, in_v2, in_v3)
    out_bufs = (out_v0, out_v1, out_v2, out_v3)

    def issue_in(t, b):
        cid = t * NW + wid
        pltpu.async_copy(tview_hbm.at[:, pl.ds(cid * 128, 128)],
                         in_bufs[b], sem_in.at[b])

    def wait_in(b):
        pltpu.make_async_copy(tview_hbm.at[:, pl.ds(0, 128)],
                              in_bufs[b], sem_in.at[b]).wait()

    def issue_out(cid, b):
        pltpu.async_copy(out_bufs[b],
                         out_hbm.at[pl.ds(cid * 2048, 2048)], sem_out.at[b])

    def wait_out(b):
        pltpu.make_async_copy(out_bufs[b],
                              out_hbm.at[pl.ds(0, 2048)], sem_out.at[b]).wait()

    for b in range(4):
        issue_in(b, b)

    def loop_body(tt, carry):
        for b in range(4):
            t = 4 * tt + b
            cid = t * NW + wid

            @pl.when(cid < NFULL)
            def _():
                wait_in(b)

                @pl.when(t >= 4)
                def _():
                    wait_out(b)

                _transpose_block(in_bufs[b], out_bufs[b])
                issue_out(cid, b)

            @pl.when((t + 4) * NW + wid < NFULL)
            def _():
                issue_in(t + 4, b)
        return carry

    lax.fori_loop(0, (NSLOT + 3) // 4, loop_body, 0)
    for b in range(4):
        wait_out(b)

    @pl.when(wid == NW - 1)
    def _():
        pltpu.sync_copy(tail_hbm, in_v0)
        _transpose_block(in_v0, out_v0)
        pltpu.sync_copy(out_v0.at[pl.ds(0, NTAIL * 16)],
                        out_hbm.at[pl.ds(NFULL * 2048, NTAIL * 16)])


def _stage2_body(ids_hbm, rat_hbm, table_hbm, bias_hbm, out_hbm,
                 idx_v, rat_v, rows_v, out_v, bias_v, sems):
    wid = lax.axis_index("s") * NC + lax.axis_index("c")

    pltpu.sync_copy(ids_hbm.at[wid], idx_v)
    pltpu.sync_copy(rat_hbm.at[wid], rat_v)
    pltpu.sync_copy(bias_hbm, bias_v)
    bvec = bias_v[...]
    zvec = jnp.zeros((16,), jnp.float32)

    def issue(chunk, buf):
        for g in range(4):
            pltpu.async_copy(
                table_hbm.at[idx_v.at[4 * chunk + g]],
                rows_v.at[pl.ds(buf * RPC + g * GROWS, GROWS)],
                sems.at[buf])

    def drain(buf):
        for g in range(4):
            pltpu.make_async_copy(
                table_hbm.at[idx_v.at[g]],
                rows_v.at[pl.ds(buf * RPC + g * GROWS, GROWS)],
                sems.at[buf]).wait()

    issue(0, 0)
    issue(1, 1)

    def compute(chunk, buf):
        base = buf * RPC

        def sample_body(s, carry2):
            gbase = (chunk * CH + s) * 32
            rv0 = rat_v[pl.ds(gbase, 16)]
            rv1 = rat_v[pl.ds(gbase + 16, 16)]
            accs = [bvec, zvec, zvec, zvec]
            for f in range(NF):
                rv = rv0 if f < 16 else rv1
                rb = _lane_broadcast(rv, f % 16)
                row = rows_v[base + s * NF + f]
                accs[f % 4] = accs[f % 4] + row * rb
            out_v[chunk * CH + s] = (accs[0] + accs[1]) + (accs[2] + accs[3])
            return carry2

        lax.fori_loop(0, CH, sample_body, 0)

    def loop_body(tt, carry):
        for b in range(2):
            c = 2 * tt + b
            drain(b)
            compute(c, b)

            @pl.when(c + 2 < NCHUNK)
            def _():
                issue(c + 2, b)
        return carry

    lax.fori_loop(0, NCHUNK // 2, loop_body, 0)
    pltpu.sync_copy(out_v, out_hbm.at[wid])


def kernel(feature_ids, feature_ratings, fc_weight, bias):
    tview = fc_weight.T
    tail128 = jnp.pad(fc_weight[NFULL * 128:].T, ((0, 0), (0, 128 - NTAIL)))

    stage1 = functools.partial(
        pl.kernel,
        out_type=jax.ShapeDtypeStruct((VOCAB * D,), jnp.float32),
        mesh=plsc.VectorSubcoreMesh(core_axis_name="c", subcore_axis_name="s"),
        compiler_params=pltpu.CompilerParams(
            use_tc_tiling_on_sc=True, needs_layout_passes=False),
        scratch_types=[
            pltpu.VMEM((16, 128), jnp.float32),
            pltpu.VMEM((16, 128), jnp.float32),
            pltpu.VMEM((16, 128), jnp.float32),
            pltpu.VMEM((16, 128), jnp.float32),
            pltpu.VMEM((2048,), jnp.float32),
            pltpu.VMEM((2048,), jnp.float32),
            pltpu.VMEM((2048,), jnp.float32),
            pltpu.VMEM((2048,), jnp.float32),
            pltpu.SemaphoreType.DMA((4,)),
            pltpu.SemaphoreType.DMA((4,)),
        ],
    )(_stage1_body)

    table_rm = stage1(tview, tail128).reshape(VOCAB, D)

    ids3 = feature_ids.reshape(NW, 4 * NCHUNK, GROWS)
    rat2 = jnp.pad(feature_ratings, ((0, 0), (0, 32 - NF))).reshape(NW, BPW * 32)

    stage2 = functools.partial(
        pl.kernel,
        out_type=jax.ShapeDtypeStruct((NW, BPW, D), jnp.float32),
        mesh=plsc.VectorSubcoreMesh(core_axis_name="c", subcore_axis_name="s"),
        compiler_params=pltpu.CompilerParams(use_tc_tiling_on_sc=False),
        scratch_types=[
            pltpu.VMEM((4 * NCHUNK, GROWS), jnp.int32),
            pltpu.VMEM((BPW * 32,), jnp.float32),
            pltpu.VMEM((2 * RPC, D), jnp.float32),
            pltpu.VMEM((BPW, D), jnp.float32),
            pltpu.VMEM((D,), jnp.float32),
            pltpu.SemaphoreType.DMA((2,)),
        ],
    )(_stage2_body)

    out = stage2(ids3, rat2, table_rm, bias)
    return out.reshape(B, D)


# speedup vs baseline: 1.6771x; 1.0040x over previous
import functools

import jax
import jax.numpy as jnp
from jax import lax
from jax.experimental import pallas as pl
from jax.experimental.pallas import tpu as pltpu
from jax.experimental.pallas import tpu_sc as plsc

VOCAB = 1000012
B = 16384
NF = 26
D = 16

NC = 2
NS = 16
NW = NC * NS
BPW = B // NW
IPW = BPW * NF

NFULL = VOCAB // 128
NTAIL = VOCAB - NFULL * 128
NSLOT = NFULL // NW + 1

CH = 16
RPC = CH * NF
GROWS = 104
NCHUNK = BPW // CH


def _lane_broadcast(vec, lane):
    idx = jnp.full((16, 1), lane, jnp.int32)
    dnums = lax.GatherDimensionNumbers(
        offset_dims=(), collapsed_slice_dims=(0,), start_index_map=(0,))
    return lax.gather(vec, idx, dnums, (1,),
                      mode=lax.GatherScatterMode.PROMISE_IN_BOUNDS)


def _transpose_block(in_ref, out_ref):
    iota = lax.broadcasted_iota(jnp.int32, (16,), 0)

    @plsc.parallel_loop(0, 128, unroll=16)
    def _(j):
        col = (iota + j) & 127
        vals = plsc.load_gather(in_ref, [iota, col])
        plsc.store_scatter(out_ref, [col * 16 + iota], vals)


def _stage1_body(tview_hbm, tail_hbm, out_hbm, in_v0, in_v1, in_v2, in_v3,
                 out_v0, out_v1, out_v2, out_v3, sem_in, sem_out):
    wid = lax.axis_index("s") * NC + lax.axis_index("c")
    in_bufs = (in_v0, in_v1, in_v2, in_v3)
    out_bufs = (out_v0, out_v1, out_v2, out_v3)

    def issue_in(t, b):
        cid = t * NW + wid
        pltpu.async_copy(tview_hbm.at[:, pl.ds(cid * 128, 128)],
                         in_bufs[b], sem_in.at[b])

    def wait_in(b):
        pltpu.make_async_copy(tview_hbm.at[:, pl.ds(0, 128)],
                              in_bufs[b], sem_in.at[b]).wait()

    def issue_out(cid, b):
        pltpu.async_copy(out_bufs[b],
                         out_hbm.at[pl.ds(cid * 2048, 2048)], sem_out.at[b])

    def wait_out(b):
        pltpu.make_async_copy(out_bufs[b],
                              out_hbm.at[pl.ds(0, 2048)], sem_out.at[b]).wait()

    for b in range(4):
        issue_in(b, b)

    def loop_body(tt, carry):
        for b in range(4):
            t = 4 * tt + b
            cid = t * NW + wid

            @pl.when(cid < NFULL)
            def _():
                wait_in(b)

                @pl.when(t >= 4)
                def _():
                    wait_out(b)

                _transpose_block(in_bufs[b], out_bufs[b])
                issue_out(cid, b)

            @pl.when((t + 4) * NW + wid < NFULL)
            def _():
                issue_in(t + 4, b)
        return carry

    lax.fori_loop(0, (NSLOT + 3) // 4, loop_body, 0)
    for b in range(4):
        wait_out(b)

    @pl.when(wid == NW - 1)
    def _():
        pltpu.sync_copy(tail_hbm, in_v0)
        _transpose_block(in_v0, out_v0)
        pltpu.sync_copy(out_v0.at[pl.ds(0, NTAIL * 16)],
                        out_hbm.at[pl.ds(NFULL * 2048, NTAIL * 16)])


def _stage2_body(ids_hbm, rat_hbm, table_hbm, bias_hbm, out_hbm,
                 idx_v, rat_v, rows_v, out_v, bias_v, sems):
    wid = lax.axis_index("s") * NC + lax.axis_index("c")

    pltpu.sync_copy(ids_hbm.at[wid], idx_v)
    pltpu.sync_copy(rat_hbm.at[wid], rat_v)
    pltpu.sync_copy(bias_hbm, bias_v)
    bvec = bias_v[...]
    zvec = jnp.zeros((16,), jnp.float32)

    def issue(chunk, buf):
        for g in range(4):
            pltpu.async_copy(
                table_hbm.at[idx_v.at[4 * chunk + g]],
                rows_v.at[pl.ds(buf * RPC + g * GROWS, GROWS)],
                sems.at[buf])

    def drain(buf):
        for g in range(4):
            pltpu.make_async_copy(
                table_hbm.at[idx_v.at[g]],
                rows_v.at[pl.ds(buf * RPC + g * GROWS, GROWS)],
                sems.at[buf]).wait()

    issue(0, 0)
    issue(1, 1)

    def compute(chunk, buf):
        base = buf * RPC

        @plsc.parallel_loop(0, CH, unroll=4)
        def _(s):
            gbase = (chunk * CH + s) * 32
            rv0 = rat_v[pl.ds(gbase, 16)]
            rv1 = rat_v[pl.ds(gbase + 16, 16)]
            accs = [bvec, zvec, zvec, zvec]
            for f in range(NF):
                rv = rv0 if f < 16 else rv1
                rb = _lane_broadcast(rv, f % 16)
                row = rows_v[base + s * NF + f]
                accs[f % 4] = accs[f % 4] + row * rb
            out_v[chunk * CH + s] = (accs[0] + accs[1]) + (accs[2] + accs[3])

    def loop_body(tt, carry):
        for b in range(2):
            c = 2 * tt + b
            drain(b)
            compute(c, b)

            @pl.when(c + 2 < NCHUNK)
            def _():
                issue(c + 2, b)
        return carry

    lax.fori_loop(0, NCHUNK // 2, loop_body, 0)
    pltpu.sync_copy(out_v, out_hbm.at[wid])


def kernel(feature_ids, feature_ratings, fc_weight, bias):
    tview = fc_weight.T
    tail128 = jnp.pad(fc_weight[NFULL * 128:].T, ((0, 0), (0, 128 - NTAIL)))

    stage1 = functools.partial(
        pl.kernel,
        out_type=jax.ShapeDtypeStruct((VOCAB * D,), jnp.float32),
        mesh=plsc.VectorSubcoreMesh(core_axis_name="c", subcore_axis_name="s"),
        compiler_params=pltpu.CompilerParams(
            use_tc_tiling_on_sc=True, needs_layout_passes=False),
        scratch_types=[
            pltpu.VMEM((16, 128), jnp.float32),
            pltpu.VMEM((16, 128), jnp.float32),
            pltpu.VMEM((16, 128), jnp.float32),
            pltpu.VMEM((16, 128), jnp.float32),
            pltpu.VMEM((2048,), jnp.float32),
            pltpu.VMEM((2048,), jnp.float32),
            pltpu.VMEM((2048,), jnp.float32),
            pltpu.VMEM((2048,), jnp.float32),
            pltpu.SemaphoreType.DMA((4,)),
            pltpu.SemaphoreType.DMA((4,)),
        ],
    )(_stage1_body)

    table_rm = stage1(tview, tail128).reshape(VOCAB, D)

    ids3 = feature_ids.reshape(NW, 4 * NCHUNK, GROWS)
    rat2 = jnp.pad(feature_ratings, ((0, 0), (0, 32 - NF))).reshape(NW, BPW * 32)

    stage2 = functools.partial(
        pl.kernel,
        out_type=jax.ShapeDtypeStruct((NW, BPW, D), jnp.float32),
        mesh=plsc.VectorSubcoreMesh(core_axis_name="c", subcore_axis_name="s"),
        compiler_params=pltpu.CompilerParams(use_tc_tiling_on_sc=False),
        scratch_types=[
            pltpu.VMEM((4 * NCHUNK, GROWS), jnp.int32),
            pltpu.VMEM((BPW * 32,), jnp.float32),
            pltpu.VMEM((2 * RPC, D), jnp.float32),
            pltpu.VMEM((BPW, D), jnp.float32),
            pltpu.VMEM((D,), jnp.float32),
            pltpu.SemaphoreType.DMA((2,)),
        ],
    )(_stage2_body)

    out = stage2(ids3, rat2, table_rm, bias)
    return out.reshape(B, D)


# speedup vs baseline: 1.7583x; 1.0484x over previous
import functools

import jax
import jax.numpy as jnp
from jax import lax
from jax.experimental import pallas as pl
from jax.experimental.pallas import tpu as pltpu
from jax.experimental.pallas import tpu_sc as plsc

VOCAB = 1000012
B = 16384
NF = 26
D = 16

NC = 2
NS = 16
NW = NC * NS
BPW = B // NW
IPW = BPW * NF

NFULL = VOCAB // 128
NTAIL = VOCAB - NFULL * 128
NSLOT = NFULL // NW + 1

CH = 16
RPC = CH * NF
GROWS = 104
NCHUNK = BPW // CH


def _lane_broadcast(vec, lane):
    idx = jnp.full((16, 1), lane, jnp.int32)
    dnums = lax.GatherDimensionNumbers(
        offset_dims=(), collapsed_slice_dims=(0,), start_index_map=(0,))
    return lax.gather(vec, idx, dnums, (1,),
                      mode=lax.GatherScatterMode.PROMISE_IN_BOUNDS)


def _transpose_block(in_ref, out_ref):
    iota = lax.broadcasted_iota(jnp.int32, (16,), 0)

    @plsc.parallel_loop(0, 128, unroll=16)
    def _(j):
        col = (iota + j) & 127
        vals = plsc.load_gather(in_ref, [iota, col])
        plsc.store_scatter(out_ref, [col * 16 + iota], vals)


def _stage1_body(tview_hbm, tail_hbm, out_hbm, in_v0, in_v1, in_v2, in_v3,
                 out_v0, out_v1, out_v2, out_v3, sem_in, sem_out):
    wid = lax.axis_index("s") * NC + lax.axis_index("c")
    in_bufs = (in_v0, in_v1, in_v2, in_v3)
    out_bufs = (out_v0, out_v1, out_v2, out_v3)

    def issue_in(t, b):
        cid = t * NW + wid
        pltpu.async_copy(tview_hbm.at[:, pl.ds(cid * 128, 128)],
                         in_bufs[b], sem_in.at[b])

    def wait_in(b):
        pltpu.make_async_copy(tview_hbm.at[:, pl.ds(0, 128)],
                              in_bufs[b], sem_in.at[b]).wait()

    def issue_out(cid, b):
        pltpu.async_copy(out_bufs[b],
                         out_hbm.at[pl.ds(cid * 2048, 2048)], sem_out.at[b])

    def wait_out(b):
        pltpu.make_async_copy(out_bufs[b],
                              out_hbm.at[pl.ds(0, 2048)], sem_out.at[b]).wait()

    for b in range(4):
        issue_in(b, b)

    def loop_body(tt, carry):
        for b in range(4):
            t = 4 * tt + b
            cid = t * NW + wid

            @pl.when(cid < NFULL)
            def _():
                wait_in(b)

                @pl.when(t >= 4)
                def _():
                    wait_out(b)

                _transpose_block(in_bufs[b], out_bufs[b])
                issue_out(cid, b)

            @pl.when((t + 4) * NW + wid < NFULL)
            def _():
                issue_in(t + 4, b)
        return carry

    lax.fori_loop(0, (NSLOT + 3) // 4, loop_body, 0)
    for b in range(4):
        wait_out(b)

    @pl.when(wid == NW - 1)
    def _():
        pltpu.sync_copy(tail_hbm, in_v0)
        _transpose_block(in_v0, out_v0)
        pltpu.sync_copy(out_v0.at[pl.ds(0, NTAIL * 16)],
                        out_hbm.at[pl.ds(NFULL * 2048, NTAIL * 16)])


def _stage2_body(ids_hbm, rat_hbm, table_hbm, bias_hbm, out_hbm,
                 idx_v, rat_v, rows_v, out_v, bias_v, sems):
    wid = lax.axis_index("s") * NC + lax.axis_index("c")

    pltpu.sync_copy(ids_hbm.at[wid], idx_v)
    pltpu.sync_copy(rat_hbm.at[wid], rat_v)
    pltpu.sync_copy(bias_hbm, bias_v)
    bvec = bias_v[...]
    zvec = jnp.zeros((16,), jnp.float32)

    def issue(chunk, buf):
        for g in range(4):
            pltpu.async_copy(
                table_hbm.at[idx_v.at[4 * chunk + g]],
                rows_v.at[pl.ds(buf * RPC + g * GROWS, GROWS)],
                sems.at[buf])

    def drain(buf):
        for g in range(4):
            pltpu.make_async_copy(
                table_hbm.at[idx_v.at[g]],
                rows_v.at[pl.ds(buf * RPC + g * GROWS, GROWS)],
                sems.at[buf]).wait()

    for bb in range(4):
        issue(bb, bb)

    def compute(chunk, buf):
        base = buf * RPC

        @plsc.parallel_loop(0, CH, unroll=4)
        def _(s):
            gbase = (chunk * CH + s) * 32
            rv0 = rat_v[pl.ds(gbase, 16)]
            rv1 = rat_v[pl.ds(gbase + 16, 16)]
            accs = [bvec, zvec, zvec, zvec]
            for f in range(NF):
                rv = rv0 if f < 16 else rv1
                rb = _lane_broadcast(rv, f % 16)
                row = rows_v[base + s * NF + f]
                accs[f % 4] = accs[f % 4] + row * rb
            out_v[chunk * CH + s] = (accs[0] + accs[1]) + (accs[2] + accs[3])

    def loop_body(tt, carry):
        for b in range(4):
            c = 4 * tt + b
            drain(b)
            compute(c, b)

            @pl.when(c + 4 < NCHUNK)
            def _():
                issue(c + 4, b)
        return carry

    lax.fori_loop(0, NCHUNK // 4, loop_body, 0)
    pltpu.sync_copy(out_v, out_hbm.at[wid])


def kernel(feature_ids, feature_ratings, fc_weight, bias):
    tview = fc_weight.T
    tail128 = jnp.pad(fc_weight[NFULL * 128:].T, ((0, 0), (0, 128 - NTAIL)))

    stage1 = functools.partial(
        pl.kernel,
        out_type=jax.ShapeDtypeStruct((VOCAB * D,), jnp.float32),
        mesh=plsc.VectorSubcoreMesh(core_axis_name="c", subcore_axis_name="s"),
        compiler_params=pltpu.CompilerParams(
            use_tc_tiling_on_sc=True, needs_layout_passes=False),
        scratch_types=[
            pltpu.VMEM((16, 128), jnp.float32),
            pltpu.VMEM((16, 128), jnp.float32),
            pltpu.VMEM((16, 128), jnp.float32),
            pltpu.VMEM((16, 128), jnp.float32),
            pltpu.VMEM((2048,), jnp.float32),
            pltpu.VMEM((2048,), jnp.float32),
            pltpu.VMEM((2048,), jnp.float32),
            pltpu.VMEM((2048,), jnp.float32),
            pltpu.SemaphoreType.DMA((4,)),
            pltpu.SemaphoreType.DMA((4,)),
        ],
    )(_stage1_body)

    table_rm = stage1(tview, tail128).reshape(VOCAB, D)

    ids3 = feature_ids.reshape(NW, 4 * NCHUNK, GROWS)
    rat2 = jnp.pad(feature_ratings, ((0, 0), (0, 32 - NF))).reshape(NW, BPW * 32)

    stage2 = functools.partial(
        pl.kernel,
        out_type=jax.ShapeDtypeStruct((NW, BPW, D), jnp.float32),
        mesh=plsc.VectorSubcoreMesh(core_axis_name="c", subcore_axis_name="s"),
        compiler_params=pltpu.CompilerParams(use_tc_tiling_on_sc=False),
        scratch_types=[
            pltpu.VMEM((4 * NCHUNK, GROWS), jnp.int32),
            pltpu.VMEM((BPW * 32,), jnp.float32),
            pltpu.VMEM((4 * RPC, D), jnp.float32),
            pltpu.VMEM((BPW, D), jnp.float32),
            pltpu.VMEM((D,), jnp.float32),
            pltpu.SemaphoreType.DMA((4,)),
        ],
    )(_stage2_body)

    out = stage2(ids3, rat2, table_rm, bias)
    return out.reshape(B, D)


# speedup vs baseline: 1.9574x; 1.1132x over previous
import functools

import jax
import jax.numpy as jnp
from jax import lax
from jax.experimental import pallas as pl
from jax.experimental.pallas import tpu as pltpu
from jax.experimental.pallas import tpu_sc as plsc

VOCAB = 1000012
B = 16384
NF = 26
D = 16

NC = 2
NS = 16
NW = NC * NS
BPW = B // NW
IPW = BPW * NF

S1W = 256
NFULL = VOCAB // S1W
NTAIL = VOCAB - NFULL * S1W
NSLOT = NFULL // NW + 1

CH = 16
RPC = CH * NF
GROWS = 104
NCHUNK = BPW // CH


def _lane_broadcast(vec, lane):
    idx = jnp.full((16, 1), lane, jnp.int32)
    dnums = lax.GatherDimensionNumbers(
        offset_dims=(), collapsed_slice_dims=(0,), start_index_map=(0,))
    return lax.gather(vec, idx, dnums, (1,),
                      mode=lax.GatherScatterMode.PROMISE_IN_BOUNDS)


def _transpose_block(in_ref, out_ref, ncols):
    iota = lax.broadcasted_iota(jnp.int32, (16,), 0)

    @plsc.parallel_loop(0, ncols, unroll=16)
    def _(j):
        sub = j >> 7
        col = sub * 128 + ((iota + (j & 127)) & 127)
        vals = plsc.load_gather(in_ref, [iota, col])
        plsc.store_scatter(out_ref, [col * 16 + iota], vals)


def _stage1_body(tview_hbm, tail_hbm, out_hbm, in_v0, in_v1, in_v2, in_v3,
                 out_v0, out_v1, out_v2, out_v3, sem_in, sem_out):
    wid = lax.axis_index("s") * NC + lax.axis_index("c")
    in_bufs = (in_v0, in_v1, in_v2, in_v3)
    out_bufs = (out_v0, out_v1, out_v2, out_v3)

    def issue_in(t, b):
        cid = t * NW + wid
        pltpu.async_copy(tview_hbm.at[:, pl.ds(cid * S1W, S1W)],
                         in_bufs[b], sem_in.at[b])

    def wait_in(b):
        pltpu.make_async_copy(tview_hbm.at[:, pl.ds(0, S1W)],
                              in_bufs[b], sem_in.at[b]).wait()

    def issue_out(cid, b):
        pltpu.async_copy(out_bufs[b],
                         out_hbm.at[pl.ds(cid * (S1W * 16), S1W * 16)],
                         sem_out.at[b])

    def wait_out(b):
        pltpu.make_async_copy(out_bufs[b],
                              out_hbm.at[pl.ds(0, S1W * 16)],
                              sem_out.at[b]).wait()

    for b in range(4):
        issue_in(b, b)

    def loop_body(tt, carry):
        for b in range(4):
            t = 4 * tt + b
            cid = t * NW + wid

            @pl.when(cid < NFULL)
            def _():
                wait_in(b)

                @pl.when(t >= 4)
                def _():
                    wait_out(b)

                _transpose_block(in_bufs[b], out_bufs[b], S1W)
                issue_out(cid, b)

            @pl.when((t + 4) * NW + wid < NFULL)
            def _():
                issue_in(t + 4, b)
        return carry

    lax.fori_loop(0, (NSLOT + 3) // 4, loop_body, 0)
    for b in range(4):
        wait_out(b)

    @pl.when(wid == NW - 1)
    def _():
        pltpu.sync_copy(tail_hbm, in_v0)
        _transpose_block(in_v0, out_v0, 128)
        pltpu.sync_copy(out_v0.at[pl.ds(0, NTAIL * 16)],
                        out_hbm.at[pl.ds(NFULL * (S1W * 16), NTAIL * 16)])


def _stage2_body(ids_hbm, rat_hbm, table_hbm, bias_hbm, out_hbm,
                 idx_v, rat_v, rows_v, out_v, bias_v, sems):
    wid = lax.axis_index("s") * NC + lax.axis_index("c")

    pltpu.sync_copy(ids_hbm.at[wid], idx_v)
    pltpu.sync_copy(rat_hbm.at[wid], rat_v)
    pltpu.sync_copy(bias_hbm, bias_v)
    bvec = bias_v[...]
    zvec = jnp.zeros((16,), jnp.float32)

    def issue(chunk, buf):
        for g in range(4):
            pltpu.async_copy(
                table_hbm.at[idx_v.at[4 * chunk + g]],
                rows_v.at[pl.ds(buf * RPC + g * GROWS, GROWS)],
                sems.at[buf])

    def drain(buf):
        for g in range(4):
            pltpu.make_async_copy(
                table_hbm.at[idx_v.at[g]],
                rows_v.at[pl.ds(buf * RPC + g * GROWS, GROWS)],
                sems.at[buf]).wait()

    for bb in range(4):
        issue(bb, bb)

    def compute(chunk, buf):
        base = buf * RPC

        @plsc.parallel_loop(0, CH, unroll=4)
        def _(s):
            gbase = (chunk * CH + s) * 32
            rv0 = rat_v[pl.ds(gbase, 16)]
            rv1 = rat_v[pl.ds(gbase + 16, 16)]
            accs = [bvec, zvec, zvec, zvec]
            for f in range(NF):
                rv = rv0 if f < 16 else rv1
                rb = _lane_broadcast(rv, f % 16)
                row = rows_v[base + s * NF + f]
                accs[f % 4] = accs[f % 4] + row * rb
            out_v[chunk * CH + s] = (accs[0] + accs[1]) + (accs[2] + accs[3])

    def loop_body(tt, carry):
        for b in range(4):
            c = 4 * tt + b
            drain(b)
            compute(c, b)

            @pl.when(c + 4 < NCHUNK)
            def _():
                issue(c + 4, b)
        return carry

    lax.fori_loop(0, NCHUNK // 4, loop_body, 0)
    pltpu.sync_copy(out_v, out_hbm.at[wid])


def kernel(feature_ids, feature_ratings, fc_weight, bias):
    tview = fc_weight.T
    tail128 = jnp.pad(fc_weight[NFULL * S1W:].T, ((0, 0), (0, S1W - NTAIL)))

    stage1 = functools.partial(
        pl.kernel,
        out_type=jax.ShapeDtypeStruct((VOCAB * D,), jnp.float32),
        mesh=plsc.VectorSubcoreMesh(core_axis_name="c", subcore_axis_name="s"),
        compiler_params=pltpu.CompilerParams(
            use_tc_tiling_on_sc=True, needs_layout_passes=False),
        scratch_types=[
            pltpu.VMEM((16, S1W), jnp.float32),
            pltpu.VMEM((16, S1W), jnp.float32),
            pltpu.VMEM((16, S1W), jnp.float32),
            pltpu.VMEM((16, S1W), jnp.float32),
            pltpu.VMEM((S1W * 16,), jnp.float32),
            pltpu.VMEM((S1W * 16,), jnp.float32),
            pltpu.VMEM((S1W * 16,), jnp.float32),
            pltpu.VMEM((S1W * 16,), jnp.float32),
            pltpu.SemaphoreType.DMA((4,)),
            pltpu.SemaphoreType.DMA((4,)),
        ],
    )(_stage1_body)

    table_rm = stage1(tview, tail128).reshape(VOCAB, D)

    ids3 = feature_ids.reshape(NW, 4 * NCHUNK, GROWS)
    rat2 = jnp.pad(feature_ratings, ((0, 0), (0, 32 - NF))).reshape(NW, BPW * 32)

    stage2 = functools.partial(
        pl.kernel,
        out_type=jax.ShapeDtypeStruct((NW, BPW, D), jnp.float32),
        mesh=plsc.VectorSubcoreMesh(core_axis_name="c", subcore_axis_name="s"),
        compiler_params=pltpu.CompilerParams(use_tc_tiling_on_sc=False),
        scratch_types=[
            pltpu.VMEM((4 * NCHUNK, GROWS), jnp.int32),
            pltpu.VMEM((BPW * 32,), jnp.float32),
            pltpu.VMEM((4 * RPC, D), jnp.float32),
            pltpu.VMEM((BPW, D), jnp.float32),
            pltpu.VMEM((D,), jnp.float32),
            pltpu.SemaphoreType.DMA((4,)),
        ],
    )(_stage2_body)

    out = stage2(ids3, rat2, table_rm, bias)
    return out.reshape(B, D)


# speedup vs baseline: 1.9691x; 1.0060x over previous
import functools

import jax
import jax.numpy as jnp
from jax import lax
from jax.experimental import pallas as pl
from jax.experimental.pallas import tpu as pltpu
from jax.experimental.pallas import tpu_sc as plsc

VOCAB = 1000012
B = 16384
NF = 26
D = 16

NC = 2
NS = 16
NW = NC * NS
BPW = B // NW
IPW = BPW * NF

S1W = 256
NFULL = VOCAB // S1W
NTAIL = VOCAB - NFULL * S1W
NSLOT = NFULL // NW + 1

CH = 16
RPC = CH * NF
GROWS = 104
NCHUNK = BPW // CH


def _lane_broadcast(vec, lane):
    idx = jnp.full((16, 1), lane, jnp.int32)
    dnums = lax.GatherDimensionNumbers(
        offset_dims=(), collapsed_slice_dims=(0,), start_index_map=(0,))
    return lax.gather(vec, idx, dnums, (1,),
                      mode=lax.GatherScatterMode.PROMISE_IN_BOUNDS)


def _transpose_block(in_ref, out_ref, out_off):
    iota = lax.broadcasted_iota(jnp.int32, (16,), 0)

    @plsc.parallel_loop(0, 128, unroll=16)
    def _(j):
        col = (iota + j) & 127
        vals = plsc.load_gather(in_ref, [iota, col])
        plsc.store_scatter(out_ref, [out_off + col * 16 + iota], vals)


def _stage1_body(tview_hbm, tail_hbm, out_hbm,
                 in_a0, in_b0, in_a1, in_b1, in_a2, in_b2, in_a3, in_b3,
                 out_v0, out_v1, out_v2, out_v3, sem_in, sem_out):
    wid = lax.axis_index("s") * NC + lax.axis_index("c")
    in_bufs = ((in_a0, in_b0), (in_a1, in_b1), (in_a2, in_b2), (in_a3, in_b3))
    out_bufs = (out_v0, out_v1, out_v2, out_v3)

    def issue_in(t, b):
        cid = t * NW + wid
        for h in range(2):
            pltpu.async_copy(
                tview_hbm.at[:, pl.ds(cid * S1W + h * 128, 128)],
                in_bufs[b][h], sem_in.at[b])

    def wait_in(b):
        for h in range(2):
            pltpu.make_async_copy(tview_hbm.at[:, pl.ds(0, 128)],
                                  in_bufs[b][h], sem_in.at[b]).wait()

    def issue_out(cid, b):
        pltpu.async_copy(out_bufs[b],
                         out_hbm.at[pl.ds(cid * (S1W * 16), S1W * 16)],
                         sem_out.at[b])

    def wait_out(b):
        pltpu.make_async_copy(out_bufs[b],
                              out_hbm.at[pl.ds(0, S1W * 16)],
                              sem_out.at[b]).wait()

    for b in range(4):
        issue_in(b, b)

    def loop_body(tt, carry):
        for b in range(4):
            t = 4 * tt + b
            cid = t * NW + wid

            @pl.when(cid < NFULL)
            def _():
                wait_in(b)

                @pl.when(t >= 4)
                def _():
                    wait_out(b)

                _transpose_block(in_bufs[b][0], out_bufs[b], 0)
                _transpose_block(in_bufs[b][1], out_bufs[b], 2048)
                issue_out(cid, b)

            @pl.when((t + 4) * NW + wid < NFULL)
            def _():
                issue_in(t + 4, b)
        return carry

    lax.fori_loop(0, (NSLOT + 3) // 4, loop_body, 0)
    for b in range(4):
        wait_out(b)

    @pl.when(wid == NW - 1)
    def _():
        pltpu.sync_copy(tail_hbm, in_a0)
        _transpose_block(in_a0, out_v0, 0)
        pltpu.sync_copy(out_v0.at[pl.ds(0, NTAIL * 16)],
                        out_hbm.at[pl.ds(NFULL * (S1W * 16), NTAIL * 16)])


def _stage2_body(ids_hbm, rat_hbm, table_hbm, bias_hbm, out_hbm,
                 idx_v, rat_v, rows_v, out_v, bias_v, sems):
    wid = lax.axis_index("s") * NC + lax.axis_index("c")

    pltpu.sync_copy(ids_hbm.at[wid], idx_v)
    pltpu.sync_copy(rat_hbm.at[wid], rat_v)
    pltpu.sync_copy(bias_hbm, bias_v)
    bvec = bias_v[...]
    zvec = jnp.zeros((16,), jnp.float32)

    def issue(chunk, buf):
        for g in range(4):
            pltpu.async_copy(
                table_hbm.at[idx_v.at[4 * chunk + g]],
                rows_v.at[pl.ds(buf * RPC + g * GROWS, GROWS)],
                sems.at[buf])

    def drain(buf):
        for g in range(4):
            pltpu.make_async_copy(
                table_hbm.at[idx_v.at[g]],
                rows_v.at[pl.ds(buf * RPC + g * GROWS, GROWS)],
                sems.at[buf]).wait()

    for bb in range(4):
        issue(bb, bb)

    def compute(chunk, buf):
        base = buf * RPC

        @plsc.parallel_loop(0, CH, unroll=4)
        def _(s):
            gbase = (chunk * CH + s) * 32
            rv0 = rat_v[pl.ds(gbase, 16)]
            rv1 = rat_v[pl.ds(gbase + 16, 16)]
            accs = [bvec, zvec, zvec, zvec]
            for f in range(NF):
                rv = rv0 if f < 16 else rv1
                rb = _lane_broadcast(rv, f % 16)
                row = rows_v[base + s * NF + f]
                accs[f % 4] = accs[f % 4] + row * rb
            out_v[chunk * CH + s] = (accs[0] + accs[1]) + (accs[2] + accs[3])

    def loop_body(tt, carry):
        for b in range(4):
            c = 4 * tt + b
            drain(b)
            compute(c, b)

            @pl.when(c + 4 < NCHUNK)
            def _():
                issue(c + 4, b)
        return carry

    lax.fori_loop(0, NCHUNK // 4, loop_body, 0)
    pltpu.sync_copy(out_v, out_hbm.at[wid])


def kernel(feature_ids, feature_ratings, fc_weight, bias):
    tview = fc_weight.T
    tail128 = jnp.pad(fc_weight[NFULL * S1W:].T, ((0, 0), (0, 128 - NTAIL)))

    stage1 = functools.partial(
        pl.kernel,
        out_type=jax.ShapeDtypeStruct((VOCAB * D,), jnp.float32),
        mesh=plsc.VectorSubcoreMesh(core_axis_name="c", subcore_axis_name="s"),
        compiler_params=pltpu.CompilerParams(
            use_tc_tiling_on_sc=True, needs_layout_passes=False),
        scratch_types=[
            pltpu.VMEM((16, 128), jnp.float32),
            pltpu.VMEM((16, 128), jnp.float32),
            pltpu.VMEM((16, 128), jnp.float32),
            pltpu.VMEM((16, 128), jnp.float32),
            pltpu.VMEM((16, 128), jnp.float32),
            pltpu.VMEM((16, 128), jnp.float32),
            pltpu.VMEM((16, 128), jnp.float32),
            pltpu.VMEM((16, 128), jnp.float32),
            pltpu.VMEM((S1W * 16,), jnp.float32),
            pltpu.VMEM((S1W * 16,), jnp.float32),
            pltpu.VMEM((S1W * 16,), jnp.float32),
            pltpu.VMEM((S1W * 16,), jnp.float32),
            pltpu.SemaphoreType.DMA((4,)),
            pltpu.SemaphoreType.DMA((4,)),
        ],
    )(_stage1_body)

    table_rm = stage1(tview, tail128).reshape(VOCAB, D)

    ids3 = feature_ids.reshape(NW, 4 * NCHUNK, GROWS)
    rat2 = jnp.pad(feature_ratings, ((0, 0), (0, 32 - NF))).reshape(NW, BPW * 32)

    stage2 = functools.partial(
        pl.kernel,
        out_type=jax.ShapeDtypeStruct((NW, BPW, D), jnp.float32),
        mesh=plsc.VectorSubcoreMesh(core_axis_name="c", subcore_axis_name="s"),
        compiler_params=pltpu.CompilerParams(use_tc_tiling_on_sc=False),
        scratch_types=[
            pltpu.VMEM((4 * NCHUNK, GROWS), jnp.int32),
            pltpu.VMEM((BPW * 32,), jnp.float32),
            pltpu.VMEM((4 * RPC, D), jnp.float32),
            pltpu.VMEM((BPW, D), jnp.float32),
            pltpu.VMEM((D,), jnp.float32),
            pltpu.SemaphoreType.DMA((4,)),
        ],
    )(_stage2_body)

    out = stage2(ids3, rat2, table_rm, bias)
    return out.reshape(B, D)


# speedup vs baseline: 2.0271x; 1.0295x over previous
import functools

import jax
import jax.numpy as jnp
from jax import lax
from jax.experimental import pallas as pl
from jax.experimental.pallas import tpu as pltpu
from jax.experimental.pallas import tpu_sc as plsc

VOCAB = 1000012
B = 16384
NF = 26
D = 16

NC = 2
NS = 16
NW = NC * NS
BPW = B // NW
IPW = BPW * NF

S1W = 512
NFULL = VOCAB // S1W
NTAIL = VOCAB - NFULL * S1W
NSLOT = NFULL // NW + 1

CH = 16
RPC = CH * NF
GROWS = 104
NCHUNK = BPW // CH


def _lane_broadcast(vec, lane):
    idx = jnp.full((16, 1), lane, jnp.int32)
    dnums = lax.GatherDimensionNumbers(
        offset_dims=(), collapsed_slice_dims=(0,), start_index_map=(0,))
    return lax.gather(vec, idx, dnums, (1,),
                      mode=lax.GatherScatterMode.PROMISE_IN_BOUNDS)


def _transpose_block(in_ref, out_ref, out_off):
    iota = lax.broadcasted_iota(jnp.int32, (16,), 0)

    @plsc.parallel_loop(0, 128, unroll=16)
    def _(j):
        col = (iota + j) & 127
        vals = plsc.load_gather(in_ref, [iota, col])
        plsc.store_scatter(out_ref, [out_off + col * 16 + iota], vals)


def _stage1_body(tview_hbm, tail_hbm, out_hbm,
                 in_a0, in_b0, in_c0, in_d0, in_a1, in_b1, in_c1, in_d1,
                 in_a2, in_b2, in_c2, in_d2, in_a3, in_b3, in_c3, in_d3,
                 out_v0, out_v1, out_v2, out_v3, sem_in, sem_out):
    wid = lax.axis_index("s") * NC + lax.axis_index("c")
    in_bufs = ((in_a0, in_b0, in_c0, in_d0), (in_a1, in_b1, in_c1, in_d1),
               (in_a2, in_b2, in_c2, in_d2), (in_a3, in_b3, in_c3, in_d3))
    out_bufs = (out_v0, out_v1, out_v2, out_v3)

    def issue_in(t, b):
        cid = t * NW + wid
        for h in range(4):
            pltpu.async_copy(
                tview_hbm.at[:, pl.ds(cid * S1W + h * 128, 128)],
                in_bufs[b][h], sem_in.at[b])

    def wait_in(b):
        for h in range(4):
            pltpu.make_async_copy(tview_hbm.at[:, pl.ds(0, 128)],
                                  in_bufs[b][h], sem_in.at[b]).wait()

    def issue_out(cid, b):
        pltpu.async_copy(out_bufs[b],
                         out_hbm.at[pl.ds(cid * (S1W * 16), S1W * 16)],
                         sem_out.at[b])

    def wait_out(b):
        pltpu.make_async_copy(out_bufs[b],
                              out_hbm.at[pl.ds(0, S1W * 16)],
                              sem_out.at[b]).wait()

    for b in range(4):
        issue_in(b, b)

    def loop_body(tt, carry):
        for b in range(4):
            t = 4 * tt + b
            cid = t * NW + wid

            @pl.when(cid < NFULL)
            def _():
                wait_in(b)

                @pl.when(t >= 4)
                def _():
                    wait_out(b)

                for h in range(4):
                    _transpose_block(in_bufs[b][h], out_bufs[b], h * 2048)
                issue_out(cid, b)

            @pl.when((t + 4) * NW + wid < NFULL)
            def _():
                issue_in(t + 4, b)
        return carry

    lax.fori_loop(0, (NSLOT + 3) // 4, loop_body, 0)
    for b in range(4):
        wait_out(b)

    @pl.when(wid == NW - 1)
    def _():
        pltpu.sync_copy(tail_hbm, in_a0)
        _transpose_block(in_a0, out_v0, 0)
        pltpu.sync_copy(out_v0.at[pl.ds(0, NTAIL * 16)],
                        out_hbm.at[pl.ds(NFULL * (S1W * 16), NTAIL * 16)])


def _stage2_body(ids_hbm, rat_hbm, table_hbm, bias_hbm, out_hbm,
                 idx_v, rat_v, rows_v, out_v, bias_v, sems):
    wid = lax.axis_index("s") * NC + lax.axis_index("c")

    pltpu.sync_copy(ids_hbm.at[wid], idx_v)
    pltpu.sync_copy(rat_hbm.at[wid], rat_v)
    pltpu.sync_copy(bias_hbm, bias_v)
    bvec = bias_v[...]
    zvec = jnp.zeros((16,), jnp.float32)

    def issue(chunk, buf):
        for g in range(4):
            pltpu.async_copy(
                table_hbm.at[idx_v.at[4 * chunk + g]],
                rows_v.at[pl.ds(buf * RPC + g * GROWS, GROWS)],
                sems.at[buf])

    def drain(buf):
        for g in range(4):
            pltpu.make_async_copy(
                table_hbm.at[idx_v.at[g]],
                rows_v.at[pl.ds(buf * RPC + g * GROWS, GROWS)],
                sems.at[buf]).wait()

    for bb in range(4):
        issue(bb, bb)

    def compute(chunk, buf):
        base = buf * RPC

        @plsc.parallel_loop(0, CH, unroll=4)
        def _(s):
            gbase = (chunk * CH + s) * 32
            rv0 = rat_v[pl.ds(gbase, 16)]
            rv1 = rat_v[pl.ds(gbase + 16, 16)]
            accs = [bvec, zvec, zvec, zvec]
            for f in range(NF):
                rv = rv0 if f < 16 else rv1
                rb = _lane_broadcast(rv, f % 16)
                row = rows_v[base + s * NF + f]
                accs[f % 4] = accs[f % 4] + row * rb
            out_v[chunk * CH + s] = (accs[0] + accs[1]) + (accs[2] + accs[3])

    def loop_body(tt, carry):
        for b in range(4):
            c = 4 * tt + b
            drain(b)
            compute(c, b)

            @pl.when(c + 4 < NCHUNK)
            def _():
                issue(c + 4, b)
        return carry

    lax.fori_loop(0, NCHUNK // 4, loop_body, 0)
    pltpu.sync_copy(out_v, out_hbm.at[wid])


def kernel(feature_ids, feature_ratings, fc_weight, bias):
    tview = fc_weight.T
    tail128 = jnp.pad(fc_weight[NFULL * S1W:].T, ((0, 0), (0, 128 - NTAIL)))

    stage1 = functools.partial(
        pl.kernel,
        out_type=jax.ShapeDtypeStruct((VOCAB * D,), jnp.float32),
        mesh=plsc.VectorSubcoreMesh(core_axis_name="c", subcore_axis_name="s"),
        compiler_params=pltpu.CompilerParams(
            use_tc_tiling_on_sc=True, needs_layout_passes=False),
        scratch_types=[
            pltpu.VMEM((16, 128), jnp.float32),
            pltpu.VMEM((16, 128), jnp.float32),
            pltpu.VMEM((16, 128), jnp.float32),
            pltpu.VMEM((16, 128), jnp.float32),
            pltpu.VMEM((16, 128), jnp.float32),
            pltpu.VMEM((16, 128), jnp.float32),
            pltpu.VMEM((16, 128), jnp.float32),
            pltpu.VMEM((16, 128), jnp.float32),
            pltpu.VMEM((16, 128), jnp.float32),
            pltpu.VMEM((16, 128), jnp.float32),
            pltpu.VMEM((16, 128), jnp.float32),
            pltpu.VMEM((16, 128), jnp.float32),
            pltpu.VMEM((16, 128), jnp.float32),
            pltpu.VMEM((16, 128), jnp.float32),
            pltpu.VMEM((16, 128), jnp.float32),
            pltpu.VMEM((16, 128), jnp.float32),
            pltpu.VMEM((S1W * 16,), jnp.float32),
            pltpu.VMEM((S1W * 16,), jnp.float32),
            pltpu.VMEM((S1W * 16,), jnp.float32),
            pltpu.VMEM((S1W * 16,), jnp.float32),
            pltpu.SemaphoreType.DMA((4,)),
            pltpu.SemaphoreType.DMA((4,)),
        ],
    )(_stage1_body)

    table_rm = stage1(tview, tail128).reshape(VOCAB, D)

    ids3 = feature_ids.reshape(NW, 4 * NCHUNK, GROWS)
    rat2 = jnp.pad(feature_ratings, ((0, 0), (0, 32 - NF))).reshape(NW, BPW * 32)

    stage2 = functools.partial(
        pl.kernel,
        out_type=jax.ShapeDtypeStruct((NW, BPW, D), jnp.float32),
        mesh=plsc.VectorSubcoreMesh(core_axis_name="c", subcore_axis_name="s"),
        compiler_params=pltpu.CompilerParams(use_tc_tiling_on_sc=False),
        scratch_types=[
            pltpu.VMEM((4 * NCHUNK, GROWS), jnp.int32),
            pltpu.VMEM((BPW * 32,), jnp.float32),
            pltpu.VMEM((4 * RPC, D), jnp.float32),
            pltpu.VMEM((BPW, D), jnp.float32),
            pltpu.VMEM((D,), jnp.float32),
            pltpu.SemaphoreType.DMA((4,)),
        ],
    )(_stage2_body)

    out = stage2(ids3, rat2, table_rm, bias)
    return out.reshape(B, D)
